# baseline (speedup 1.0000x reference)
import functools

import jax
import jax.numpy as jnp
import numpy as np
from jax import lax
from jax.experimental import pallas as pl
from jax.experimental.pallas import tpu as pltpu
from jax.experimental.pallas import tpu_sc as plsc

_VOCAB = 1000000
_D = 64
_B = 4096
_L = 200
_N = _B * _L

_NC = 2
_NS = 16
_NW = _NC * _NS
_PER_W = _N // _NW
_STREAM = 128
_SPC = 4
_CH = _STREAM * _SPC
_NCHUNK = _PER_W // _CH


def _make_pe(d_model: int, max_len: int) -> np.ndarray:
    position = np.arange(max_len, dtype=np.float32)[:, None]
    div_term = np.exp(
        np.arange(0, d_model, 2, dtype=np.float32) * (-np.log(10000.0) / d_model)
    )
    pe = np.zeros((max_len, d_model), dtype=np.float32)
    pe[:, 0::2] = np.sin(position * div_term)
    pe[:, 1::2] = np.cos(position * div_term)
    return pe


_PE = _make_pe(_D, _L)


def _sc_embed(W, xf, pe):
    mesh = plsc.VectorSubcoreMesh(core_axis_name="c", subcore_axis_name="s")

    @functools.partial(
        pl.kernel,
        out_type=jax.ShapeDtypeStruct((_N, _D), jnp.float32),
        mesh=mesh,
        scratch_types=[
            pltpu.VMEM((_SPC, _STREAM), jnp.int32),
            pltpu.VMEM((_CH, _D), jnp.float32),
            pltpu.VMEM((_L, _D), jnp.float32),
            pltpu.SemaphoreType.DMA,
        ],
        compiler_params=pltpu.CompilerParams(use_tc_tiling_on_sc=False),
    )
    def body(w_hbm, xf_hbm, pe_hbm, out_hbm, idx_v, rows_v, pe_v, gsem):
        wid = lax.axis_index("s") * _NC + lax.axis_index("c")
        base = wid * _PER_W
        pltpu.sync_copy(pe_hbm, pe_v)

        @pl.loop(0, _NCHUNK)
        def _chunk(g):
            off = base + g * _CH
            for j in range(_SPC):
                pltpu.sync_copy(
                    xf_hbm.at[pl.ds(off + j * _STREAM, _STREAM)], idx_v.at[j]
                )
            copies = [
                pltpu.async_copy(
                    w_hbm.at[idx_v.at[j]],
                    rows_v.at[pl.ds(j * _STREAM, _STREAM)],
                    gsem,
                )
                for j in range(_SPC)
            ]
            for c in copies:
                c.wait()

            @pl.loop(0, _CH)
            def _row(r):
                p = lax.rem(off + r, _L)
                for u in range(_D // 16):
                    v = pe_v[p, pl.ds(u * 16, 16)]
                    plsc.addupdate(rows_v.at[r, pl.ds(u * 16, 16)], v)

            pltpu.sync_copy(rows_v, out_hbm.at[pl.ds(off, _CH)])

    return body(W, xf, pe)


def kernel(x, W):
    xf = x.reshape(-1).astype(jnp.int32)
    pe = jnp.asarray(_PE)
    out = _sc_embed(W, xf, pe)
    return out.reshape(_B, _L, _D)


# speedup vs baseline: 1.1802x; 1.1802x over previous
import functools

import jax
import jax.numpy as jnp
import numpy as np
from jax import lax
from jax.experimental import pallas as pl
from jax.experimental.pallas import tpu as pltpu
from jax.experimental.pallas import tpu_sc as plsc

_VOCAB = 1000000
_D = 64
_B = 4096
_L = 200
_N = _B * _L

_NC = 2
_NS = 16
_NW = _NC * _NS
_PER_W = _N // _NW
_STREAM = 128
_SPC = 2
_CH = _STREAM * _SPC
_NCHUNK = _PER_W // _CH
_NB = 4


def _make_pe(d_model: int, max_len: int) -> np.ndarray:
    position = np.arange(max_len, dtype=np.float32)[:, None]
    div_term = np.exp(
        np.arange(0, d_model, 2, dtype=np.float32) * (-np.log(10000.0) / d_model)
    )
    pe = np.zeros((max_len, d_model), dtype=np.float32)
    pe[:, 0::2] = np.sin(position * div_term)
    pe[:, 1::2] = np.cos(position * div_term)
    return pe


_PE = _make_pe(_D, _L)


def _sc_embed(W, xf3, pe):
    mesh = plsc.VectorSubcoreMesh(core_axis_name="c", subcore_axis_name="s")

    @functools.partial(
        pl.kernel,
        out_type=jax.ShapeDtypeStruct((_N, _D), jnp.float32),
        mesh=mesh,
        scratch_types=[
            pltpu.VMEM((_PER_W // _STREAM, _STREAM), jnp.int32),
            pltpu.VMEM((_NB, _CH, _D), jnp.float32),
            pltpu.VMEM((_L, _D), jnp.float32),
            pltpu.SemaphoreType.DMA,
            pltpu.SemaphoreType.DMA,
            pltpu.SemaphoreType.DMA,
            pltpu.SemaphoreType.DMA,
            pltpu.SemaphoreType.DMA,
            pltpu.SemaphoreType.DMA,
            pltpu.SemaphoreType.DMA,
            pltpu.SemaphoreType.DMA,
        ],
        compiler_params=pltpu.CompilerParams(use_tc_tiling_on_sc=False),
    )
    def body(w_hbm, xf3_hbm, pe_hbm, out_hbm, idx_all, rows, pe_v,
             g0, g1, g2, g3, o0, o1, o2, o3):
        gs = (g0, g1, g2, g3)
        os_ = (o0, o1, o2, o3)
        wid = lax.axis_index("s") * _NC + lax.axis_index("c")
        base = wid * _PER_W
        pltpu.sync_copy(pe_hbm, pe_v)
        pltpu.sync_copy(xf3_hbm.at[wid], idx_all)

        def fire_gather(i, b):
            for j in range(_SPC):
                pltpu.async_copy(
                    w_hbm.at[idx_all.at[i * _SPC + j]],
                    rows.at[b, pl.ds(j * _STREAM, _STREAM)],
                    gs[b],
                )

        def wait_gather(b):
            for j in range(_SPC):
                pltpu.make_async_copy(
                    w_hbm.at[idx_all.at[j]],
                    rows.at[b, pl.ds(j * _STREAM, _STREAM)],
                    gs[b],
                ).wait()

        def drain_store(b):
            pltpu.make_async_copy(
                rows.at[b], out_hbm.at[pl.ds(base, _CH)], os_[b]
            ).wait()

        fire_gather(0, 0)
        fire_gather(1, 1)

        @pl.loop(0, _NCHUNK, step=_NB)
        def _super(g):
            for b in range(_NB):
                i = g + b
                wait_gather(b)

                @pl.loop(0, _CH, unroll=8)
                def _row(r):
                    p = lax.rem(i * _CH + r, _L)
                    for u in range(_D // 16):
                        plsc.addupdate(
                            rows.at[b, r, pl.ds(u * 16, 16)],
                            pe_v[p, pl.ds(u * 16, 16)],
                        )

                pltpu.async_copy(
                    rows.at[b], out_hbm.at[pl.ds(base + i * _CH, _CH)], os_[b]
                )

                nb = (b + 2) % _NB

                @pl.when(i + 2 < _NCHUNK)
                def _():
                    @pl.when(i >= 2)
                    def _():
                        drain_store(nb)

                    fire_gather(i + 2, nb)

        for b in range(_NB):
            drain_store(b)

    return body(W, xf3, pe)


def kernel(x, W):
    xf3 = x.astype(jnp.int32).reshape(_NW, _PER_W // _STREAM, _STREAM)
    pe = jnp.asarray(_PE)
    out = _sc_embed(W, xf3, pe)
    return out.reshape(_B, _L, _D)
